# baseline (speedup 1.0000x reference)
import functools

import jax
import jax.numpy as jnp
from jax import lax
from jax.experimental import pallas as pl
from jax.experimental.pallas import tpu as pltpu
from jax.experimental.pallas import tpu_sc as plsc

NN = 10000
DD = 128
EE = 320000

NC = 2
NS = 16
LL = 16

NW = NC * NS
EPW = EE // NW
CHUNK = 120
NCHUNKS = 84
EPWP = NCHUNKS * CHUNK
NPAD = 10240
RPT = NPAD // NS

_mesh = plsc.VectorSubcoreMesh(
    core_axis_name="c", subcore_axis_name="s", num_cores=NC, num_subcores=NS)
_sc_params = pltpu.CompilerParams(needs_layout_passes=False)


@functools.partial(
    pl.kernel,
    out_type=jax.ShapeDtypeStruct((NC, NPAD), jnp.float32),
    mesh=_mesh,
    compiler_params=_sc_params,
    scratch_types=[
        pltpu.VMEM((EPW,), jnp.int32),
        pltpu.VMEM((EPW,), jnp.float32),
        pltpu.VMEM((NPAD,), jnp.float32),
        pltpu.VMEM((NS, RPT), jnp.float32),
        pltpu.VMEM((RPT,), jnp.float32),
        pltpu.VMEM_SHARED((NS, NPAD), jnp.float32),
    ],
)
def _deg_kernel(dst_hbm, ew_hbm, out_hbm, dst_v, ew_v, deg_v, red_v, outb_v,
                shared):
    c = lax.axis_index("c")
    s = lax.axis_index("s")
    wid = c * NS + s

    def zero(i, _):
        deg_v[pl.ds(i * LL, LL)] = jnp.zeros((LL,), jnp.float32)
        return 0
    lax.fori_loop(0, NPAD // LL, zero, 0)

    pltpu.sync_copy(dst_hbm.at[pl.ds(wid * EPW, EPW)], dst_v)
    pltpu.sync_copy(ew_hbm.at[pl.ds(wid * EPW, EPW)], ew_v)

    def accum(i, _):
        idx = dst_v[pl.ds(i * LL, LL)]
        w = ew_v[pl.ds(i * LL, LL)]
        plsc.addupdate_scatter(deg_v, [idx], w)
        return 0
    lax.fori_loop(0, EPW // LL, accum, 0)

    pltpu.sync_copy(deg_v, shared.at[s])
    plsc.subcore_barrier()

    base = s * RPT
    pltpu.sync_copy(shared.at[:, pl.ds(base, RPT)], red_v)

    def reduce_vreg(j, _):
        acc = red_v[0, pl.ds(j * LL, LL)]
        for p in range(1, NS):
            acc = acc + red_v[p, pl.ds(j * LL, LL)]
        outb_v[pl.ds(j * LL, LL)] = acc
        return 0
    lax.fori_loop(0, RPT // LL, reduce_vreg, 0)
    pltpu.sync_copy(outb_v, out_hbm.at[c, pl.ds(base, RPT)])


@functools.partial(
    pl.kernel,
    out_type=jax.ShapeDtypeStruct((NC, NPAD, DD), jnp.float32),
    mesh=_mesh,
    compiler_params=_sc_params,
    scratch_types=[
        pltpu.VMEM((2, CHUNK), jnp.int32),
        pltpu.VMEM((2, CHUNK), jnp.int32),
        pltpu.VMEM((2, CHUNK), jnp.float32),
        pltpu.VMEM((CHUNK, DD), jnp.float32),
        pltpu.VMEM((CHUNK, DD), jnp.float32),
        pltpu.VMEM((CHUNK, DD), jnp.float32),
        pltpu.VMEM_SHARED((NPAD, DD), jnp.float32),
        pltpu.SemaphoreType.DMA,
        pltpu.SemaphoreType.DMA,
        pltpu.SemaphoreType.DMA,
    ],
)
def _agg_kernel(hs_hbm, src_hbm, dst_hbm, ew_hbm, out_hbm,
                src_t, dst_t, ew_t, gbuf_a, gbuf_b, sbuf,
                acc, sem_ga, sem_gb, sem_s):
    c = lax.axis_index("c")
    s = lax.axis_index("s")
    wid = c * NS + s

    def zfill(i, _):
        r = i // (DD // LL)
        d = i % (DD // LL)
        sbuf[r, pl.ds(d * LL, LL)] = jnp.zeros((LL,), jnp.float32)
        return 0
    lax.fori_loop(0, CHUNK * (DD // LL), zfill, 0)

    rbase = s * RPT

    def zacc(i, _):
        pltpu.sync_copy(sbuf, acc.at[pl.ds(rbase + i * CHUNK, CHUNK)])
        return 0
    lax.fori_loop(0, RPT // CHUNK, zacc, 0)
    pltpu.sync_copy(sbuf.at[pl.ds(0, RPT - (RPT // CHUNK) * CHUNK)],
                    acc.at[pl.ds(rbase + (RPT // CHUNK) * CHUNK,
                                 RPT - (RPT // CHUNK) * CHUNK)])
    plsc.subcore_barrier()

    def load_slot(ci, k):
        row = wid * NCHUNKS + ci
        pltpu.sync_copy(src_hbm.at[row], src_t.at[k])
        pltpu.sync_copy(dst_hbm.at[row], dst_t.at[k])
        pltpu.sync_copy(ew_hbm.at[row], ew_t.at[k])

    def start_gather(k, gbuf, sem):
        pltpu.async_copy(hs_hbm.at[src_t.at[k]], gbuf, sem)

    def wait_gather(gbuf, sem):
        pltpu.make_async_copy(hs_hbm.at[pl.ds(0, CHUNK)], gbuf, sem).wait()

    def start_scatter(k):
        pltpu.async_copy(sbuf, acc.at[dst_t.at[k]], sem_s, add=True)

    def wait_scatter():
        pltpu.make_async_copy(sbuf, acc.at[pl.ds(0, CHUNK)], sem_s).wait()

    def scale(k, gbuf):
        kvec = jnp.full((LL,), k, jnp.int32)

        @plsc.parallel_loop(0, CHUNK, unroll=4)
        def _(e):
            b = plsc.load_gather(ew_t, [kvec, jnp.full((LL,), e, jnp.int32)])
            for d in range(DD // LL):
                sbuf[e, pl.ds(d * LL, LL)] = gbuf[e, pl.ds(d * LL, LL)] * b

    def step(ci, slot, gbuf, gsem, gbuf_o, gsem_o, first=False, last=False):
        wait_gather(gbuf, gsem)
        if not first:
            wait_scatter()
        if not last:
            load_slot(ci + 1, 1 - slot)
            start_gather(1 - slot, gbuf_o, gsem_o)
        scale(slot, gbuf)
        start_scatter(slot)

    load_slot(0, 0)
    start_gather(0, gbuf_a, sem_ga)
    step(0, 0, gbuf_a, sem_ga, gbuf_b, sem_gb, first=True)

    def pair(i, _):
        step(2 * i + 1, 1, gbuf_b, sem_gb, gbuf_a, sem_ga)
        step(2 * i + 2, 0, gbuf_a, sem_ga, gbuf_b, sem_gb)
        return 0
    lax.fori_loop(0, (NCHUNKS - 2) // 2, pair, 0)

    step(NCHUNKS - 1, 1, gbuf_b, sem_gb, gbuf_a, sem_ga, last=True)
    wait_scatter()

    plsc.subcore_barrier()
    pltpu.sync_copy(acc.at[pl.ds(rbase, RPT)], out_hbm.at[c, pl.ds(rbase, RPT)])


RB = 400
GRID = NN // RB


def _tc1_body(x_ref, w_ref, d0_ref, d1_ref, o_ref):
    deg = d0_ref[...] + d1_ref[...] + 1.0
    dis = lax.rsqrt(deg)
    h = jnp.dot(x_ref[...], w_ref[...], preferred_element_type=jnp.float32)
    o_ref[...] = h * dis


def _tc2_body(p0_ref, p1_ref, hs_ref, d0_ref, d1_ref, w_ref, b_ref, a_ref,
              o_ref):
    deg = d0_ref[...] + d1_ref[...] + 1.0
    dis = lax.rsqrt(deg)
    t = dis * (p0_ref[...] + p1_ref[...] + hs_ref[...]) + b_ref[...]
    z = jnp.where(t > 0, t, a_ref[...] * t)
    m = jnp.dot(z, w_ref[...], preferred_element_type=jnp.float32)
    o_ref[...] = m * dis


def _tc3_body(p0_ref, p1_ref, hs_ref, d0_ref, d1_ref, b_ref, a_ref, o_ref):
    deg = d0_ref[...] + d1_ref[...] + 1.0
    dis = lax.rsqrt(deg)
    t = dis * (p0_ref[...] + p1_ref[...] + hs_ref[...]) + b_ref[...]
    o_ref[...] = jnp.where(t > 0, t, a_ref[...] * t)


_row_spec = pl.BlockSpec((RB, DD), lambda i: (i, 0))
_col_spec = pl.BlockSpec((RB, 1), lambda i: (i, 0))
_w_spec = pl.BlockSpec((DD, DD), lambda i: (0, 0))
_vec_spec = pl.BlockSpec((1, DD), lambda i: (0, 0))
_out_sds = jax.ShapeDtypeStruct((NN, DD), jnp.float32)

_tc1 = pl.pallas_call(
    _tc1_body, grid=(GRID,),
    in_specs=[_row_spec, _w_spec, _col_spec, _col_spec],
    out_specs=_row_spec, out_shape=_out_sds)

_tc2 = pl.pallas_call(
    _tc2_body, grid=(GRID,),
    in_specs=[_row_spec, _row_spec, _row_spec, _col_spec, _col_spec,
              _w_spec, _vec_spec, _vec_spec],
    out_specs=_row_spec, out_shape=_out_sds)

_tc3 = pl.pallas_call(
    _tc3_body, grid=(GRID,),
    in_specs=[_row_spec, _row_spec, _row_spec, _col_spec, _col_spec,
              _vec_spec, _vec_spec],
    out_specs=_row_spec, out_shape=_out_sds)


def kernel(x, edge_index, edge_weights, W1, b1, W2, b2, alpha):
    src = edge_index[0].astype(jnp.int32)
    dst = edge_index[1].astype(jnp.int32)
    ew = edge_weights.astype(jnp.float32)

    padw = ((0, 0), (0, EPWP - EPW))
    srcr = jnp.pad(src.reshape(NW, EPW), padw).reshape(NW * NCHUNKS, CHUNK)
    dstr = jnp.pad(dst.reshape(NW, EPW), padw).reshape(NW * NCHUNKS, CHUNK)
    ewr = jnp.pad(ew.reshape(NW, EPW), padw).reshape(NW * NCHUNKS, CHUNK)

    degp = _deg_kernel(dst, ew)
    d0 = degp[0, :NN].reshape(NN, 1)
    d1 = degp[1, :NN].reshape(NN, 1)
    b1r = b1.reshape(1, DD)
    b2r = b2.reshape(1, DD)
    ar = alpha.reshape(1, DD)

    hs1 = _tc1(x, W1, d0, d1)
    p = _agg_kernel(hs1, srcr, dstr, ewr)
    hs2 = _tc2(p[0, :NN], p[1, :NN], hs1, d0, d1, W2, b1r, ar)
    q = _agg_kernel(hs2, srcr, dstr, ewr)
    out = _tc3(q[0, :NN], q[1, :NN], hs2, d0, d1, b2r, ar)
    return out


# speedup vs baseline: 1.6573x; 1.6573x over previous
import functools

import jax
import jax.numpy as jnp
from jax import lax
from jax.experimental import pallas as pl
from jax.experimental.pallas import tpu as pltpu
from jax.experimental.pallas import tpu_sc as plsc

NN = 10000
DD = 128
EE = 320000

NC = 2
NS = 16
LL = 16

NW = NC * NS
EPW = EE // NW
CHUNK = 80
NCHUNKS = EPW // CHUNK
NPAD = 10240
RPT = NPAD // NS

_mesh = plsc.VectorSubcoreMesh(
    core_axis_name="c", subcore_axis_name="s", num_cores=NC, num_subcores=NS)
_sc_params = pltpu.CompilerParams(needs_layout_passes=False)


@functools.partial(
    pl.kernel,
    out_type=jax.ShapeDtypeStruct((NC, NPAD), jnp.float32),
    mesh=_mesh,
    compiler_params=_sc_params,
    scratch_types=[
        pltpu.VMEM((EPW,), jnp.int32),
        pltpu.VMEM((EPW,), jnp.float32),
        pltpu.VMEM((NPAD,), jnp.float32),
        pltpu.VMEM((NS, RPT), jnp.float32),
        pltpu.VMEM((RPT,), jnp.float32),
        pltpu.VMEM_SHARED((NS, NPAD), jnp.float32),
    ],
)
def _deg_kernel(dst_hbm, ew_hbm, out_hbm, dst_v, ew_v, deg_v, red_v, outb_v,
                shared):
    c = lax.axis_index("c")
    s = lax.axis_index("s")
    wid = c * NS + s

    def zero(i, _):
        deg_v[pl.ds(i * LL, LL)] = jnp.zeros((LL,), jnp.float32)
        return 0
    lax.fori_loop(0, NPAD // LL, zero, 0)

    pltpu.sync_copy(dst_hbm.at[pl.ds(wid * EPW, EPW)], dst_v)
    pltpu.sync_copy(ew_hbm.at[pl.ds(wid * EPW, EPW)], ew_v)

    def accum(i, _):
        idx = dst_v[pl.ds(i * LL, LL)]
        w = ew_v[pl.ds(i * LL, LL)]
        plsc.addupdate_scatter(deg_v, [idx], w)
        return 0
    lax.fori_loop(0, EPW // LL, accum, 0)

    pltpu.sync_copy(deg_v, shared.at[s])
    plsc.subcore_barrier()

    base = s * RPT
    pltpu.sync_copy(shared.at[:, pl.ds(base, RPT)], red_v)

    def reduce_vreg(j, _):
        acc = red_v[0, pl.ds(j * LL, LL)]
        for p in range(1, NS):
            acc = acc + red_v[p, pl.ds(j * LL, LL)]
        outb_v[pl.ds(j * LL, LL)] = acc
        return 0
    lax.fori_loop(0, RPT // LL, reduce_vreg, 0)
    pltpu.sync_copy(outb_v, out_hbm.at[c, pl.ds(base, RPT)])


@functools.partial(
    pl.kernel,
    out_type=jax.ShapeDtypeStruct((NC, NPAD, DD), jnp.float32),
    mesh=_mesh,
    compiler_params=_sc_params,
    scratch_types=[
        pltpu.VMEM((32, CHUNK), jnp.int32),
        pltpu.VMEM((CHUNK, DD), jnp.float32),
        pltpu.VMEM((CHUNK, DD), jnp.float32),
        pltpu.VMEM((CHUNK, DD), jnp.float32),
        pltpu.VMEM((CHUNK, DD), jnp.float32),
        pltpu.VMEM_SHARED((NPAD, DD), jnp.float32),
        pltpu.SemaphoreType.DMA,
        pltpu.SemaphoreType.DMA,
        pltpu.SemaphoreType.DMA,
        pltpu.SemaphoreType.DMA,
    ],
)
def _agg_kernel(hs_hbm, ed_hbm, out_hbm,
                ed_t, gbuf_a, gbuf_b, sbuf_c, sbuf_d,
                acc, sem_ga, sem_gb, sem_sc, sem_sd):
    c = lax.axis_index("c")
    s = lax.axis_index("s")
    wid = c * NS + s

    def zfill(i, _):
        r = i // (DD // LL)
        d = i % (DD // LL)
        sbuf_c[r, pl.ds(d * LL, LL)] = jnp.zeros((LL,), jnp.float32)
        return 0
    lax.fori_loop(0, CHUNK * (DD // LL), zfill, 0)

    rbase = s * RPT

    def zacc(i, _):
        pltpu.sync_copy(sbuf_c, acc.at[pl.ds(rbase + i * CHUNK, CHUNK)])
        return 0
    lax.fori_loop(0, RPT // CHUNK, zacc, 0)
    plsc.subcore_barrier()

    def load_slot(ci, k):
        row = wid * NCHUNKS + ci
        pltpu.sync_copy(ed_hbm.at[pl.ds(8 * row, 8)],
                        ed_t.at[pl.ds(8 * k, 8)])

    def start_gather(k, gbuf, sem):
        pltpu.async_copy(hs_hbm.at[ed_t.at[8 * k]], gbuf, sem)

    def wait_gather(gbuf, sem):
        pltpu.make_async_copy(hs_hbm.at[pl.ds(0, CHUNK)], gbuf, sem).wait()

    def start_scatter(k, sbuf, sem):
        pltpu.async_copy(sbuf, acc.at[ed_t.at[8 * k + 1]], sem, add=True)

    def wait_scatter(sbuf, sem):
        pltpu.make_async_copy(sbuf, acc.at[pl.ds(0, CHUNK)], sem).wait()

    def scale(k, gbuf, sbuf):
        rvec = jnp.full((LL,), 8 * k + 2, jnp.int32)

        @plsc.parallel_loop(0, CHUNK, unroll=4)
        def _(e):
            b = plsc.bitcast(
                plsc.load_gather(ed_t, [rvec, jnp.full((LL,), e, jnp.int32)]),
                jnp.float32)
            for d in range(DD // LL):
                sbuf[e, pl.ds(d * LL, LL)] = gbuf[e, pl.ds(d * LL, LL)] * b

    def step(ci, k_use, k_load, gbuf, sbuf, gsem, ssem):
        wait_gather(gbuf, gsem)

        @pl.when(ci >= 2)
        def _():
            wait_scatter(sbuf, ssem)

        @pl.when(ci + 2 < NCHUNKS)
        def _():
            load_slot(ci + 2, k_load)

        scale(k_use, gbuf, sbuf)

        @pl.when(ci + 2 < NCHUNKS)
        def _():
            start_gather(k_load, gbuf, gsem)

        start_scatter(k_use, sbuf, ssem)

    load_slot(0, 0)
    start_gather(0, gbuf_a, sem_ga)
    load_slot(1, 1)
    start_gather(1, gbuf_b, sem_gb)

    def quad(q, _):
        ci = 4 * q
        step(ci, 0, 2, gbuf_a, sbuf_c, sem_ga, sem_sc)
        step(ci + 1, 1, 3, gbuf_b, sbuf_d, sem_gb, sem_sd)
        step(ci + 2, 2, 0, gbuf_a, sbuf_c, sem_ga, sem_sc)
        step(ci + 3, 3, 1, gbuf_b, sbuf_d, sem_gb, sem_sd)
        return 0
    lax.fori_loop(0, NCHUNKS // 4, quad, 0)

    step(NCHUNKS - 1, 0, 2, gbuf_a, sbuf_c, sem_ga, sem_sc)
    wait_scatter(sbuf_c, sem_sc)
    wait_scatter(sbuf_d, sem_sd)

    plsc.subcore_barrier()
    pltpu.sync_copy(acc.at[pl.ds(rbase, RPT)], out_hbm.at[c, pl.ds(rbase, RPT)])


RB = 400
GRID = NN // RB


def _tc1_body(x_ref, w_ref, d0_ref, d1_ref, o_ref):
    deg = d0_ref[...] + d1_ref[...] + 1.0
    dis = lax.rsqrt(deg)
    h = jnp.dot(x_ref[...], w_ref[...], preferred_element_type=jnp.float32)
    o_ref[...] = h * dis


def _tc2_body(p0_ref, p1_ref, hs_ref, d0_ref, d1_ref, w_ref, b_ref, a_ref,
              o_ref):
    deg = d0_ref[...] + d1_ref[...] + 1.0
    dis = lax.rsqrt(deg)
    t = dis * (p0_ref[...] + p1_ref[...] + hs_ref[...]) + b_ref[...]
    z = jnp.where(t > 0, t, a_ref[...] * t)
    m = jnp.dot(z, w_ref[...], preferred_element_type=jnp.float32)
    o_ref[...] = m * dis


def _tc3_body(p0_ref, p1_ref, hs_ref, d0_ref, d1_ref, b_ref, a_ref, o_ref):
    deg = d0_ref[...] + d1_ref[...] + 1.0
    dis = lax.rsqrt(deg)
    t = dis * (p0_ref[...] + p1_ref[...] + hs_ref[...]) + b_ref[...]
    o_ref[...] = jnp.where(t > 0, t, a_ref[...] * t)


_row_spec = pl.BlockSpec((RB, DD), lambda i: (i, 0))
_col_spec = pl.BlockSpec((RB, 1), lambda i: (i, 0))
_w_spec = pl.BlockSpec((DD, DD), lambda i: (0, 0))
_vec_spec = pl.BlockSpec((1, DD), lambda i: (0, 0))
_out_sds = jax.ShapeDtypeStruct((NN, DD), jnp.float32)

_tc1 = pl.pallas_call(
    _tc1_body, grid=(GRID,),
    in_specs=[_row_spec, _w_spec, _col_spec, _col_spec],
    out_specs=_row_spec, out_shape=_out_sds)

_tc2 = pl.pallas_call(
    _tc2_body, grid=(GRID,),
    in_specs=[_row_spec, _row_spec, _row_spec, _col_spec, _col_spec,
              _w_spec, _vec_spec, _vec_spec],
    out_specs=_row_spec, out_shape=_out_sds)

_tc3 = pl.pallas_call(
    _tc3_body, grid=(GRID,),
    in_specs=[_row_spec, _row_spec, _row_spec, _col_spec, _col_spec,
              _vec_spec, _vec_spec],
    out_specs=_row_spec, out_shape=_out_sds)


def kernel(x, edge_index, edge_weights, W1, b1, W2, b2, alpha):
    src = edge_index[0].astype(jnp.int32)
    dst = edge_index[1].astype(jnp.int32)
    ew = edge_weights.astype(jnp.float32)

    ed = jnp.pad(
        jnp.stack([src.reshape(NW * NCHUNKS, CHUNK),
                   dst.reshape(NW * NCHUNKS, CHUNK),
                   lax.bitcast_convert_type(ew, jnp.int32)
                      .reshape(NW * NCHUNKS, CHUNK)], axis=1),
        ((0, 0), (0, 5), (0, 0))).reshape(NW * NCHUNKS * 8, CHUNK)

    degp = _deg_kernel(dst, ew)
    d0 = degp[0, :NN].reshape(NN, 1)
    d1 = degp[1, :NN].reshape(NN, 1)
    b1r = b1.reshape(1, DD)
    b2r = b2.reshape(1, DD)
    ar = alpha.reshape(1, DD)

    hs1 = _tc1(x, W1, d0, d1)
    p = _agg_kernel(hs1, ed)
    hs2 = _tc2(p[0, :NN], p[1, :NN], hs1, d0, d1, W2, b1r, ar)
    q = _agg_kernel(hs2, ed)
    out = _tc3(q[0, :NN], q[1, :NN], hs2, d0, d1, b2r, ar)
    return out


# speedup vs baseline: 1.8124x; 1.0936x over previous
import functools

import jax
import jax.numpy as jnp
from jax import lax
from jax.experimental import pallas as pl
from jax.experimental.pallas import tpu as pltpu
from jax.experimental.pallas import tpu_sc as plsc

NN = 10000
DD = 128
EE = 320000

NC = 2
NS = 16
LL = 16

NW = NC * NS
EPW = EE // NW
CHUNK = 80
NCHUNKS = EPW // CHUNK
NPAD = 10240
RPT = NPAD // NS

_mesh = plsc.VectorSubcoreMesh(
    core_axis_name="c", subcore_axis_name="s", num_cores=NC, num_subcores=NS)
_sc_params = pltpu.CompilerParams(needs_layout_passes=False)


@functools.partial(
    pl.kernel,
    out_type=jax.ShapeDtypeStruct((NC, NPAD), jnp.float32),
    mesh=_mesh,
    compiler_params=_sc_params,
    scratch_types=[
        pltpu.VMEM((EPW,), jnp.int32),
        pltpu.VMEM((EPW,), jnp.float32),
        pltpu.VMEM((NPAD,), jnp.float32),
        pltpu.VMEM((NS, RPT), jnp.float32),
        pltpu.VMEM((RPT,), jnp.float32),
        pltpu.VMEM_SHARED((NS, NPAD), jnp.float32),
    ],
)
def _deg_kernel(dst_hbm, ew_hbm, out_hbm, dst_v, ew_v, deg_v, red_v, outb_v,
                shared):
    c = lax.axis_index("c")
    s = lax.axis_index("s")
    wid = c * NS + s

    def zero(i, _):
        deg_v[pl.ds(i * LL, LL)] = jnp.zeros((LL,), jnp.float32)
        return 0
    lax.fori_loop(0, NPAD // LL, zero, 0)

    pltpu.sync_copy(dst_hbm.at[pl.ds(wid * EPW, EPW)], dst_v)
    pltpu.sync_copy(ew_hbm.at[pl.ds(wid * EPW, EPW)], ew_v)

    def accum(i, _):
        idx = dst_v[pl.ds(i * LL, LL)]
        w = ew_v[pl.ds(i * LL, LL)]
        plsc.addupdate_scatter(deg_v, [idx], w)
        return 0
    lax.fori_loop(0, EPW // LL, accum, 0)

    pltpu.sync_copy(deg_v, shared.at[s])
    plsc.subcore_barrier()

    base = s * RPT
    pltpu.sync_copy(shared.at[:, pl.ds(base, RPT)], red_v)

    def reduce_vreg(j, _):
        acc = red_v[0, pl.ds(j * LL, LL)]
        for p in range(1, NS):
            acc = acc + red_v[p, pl.ds(j * LL, LL)]
        outb_v[pl.ds(j * LL, LL)] = acc
        return 0
    lax.fori_loop(0, RPT // LL, reduce_vreg, 0)
    pltpu.sync_copy(outb_v, out_hbm.at[c, pl.ds(base, RPT)])


@functools.partial(
    pl.kernel,
    out_type=jax.ShapeDtypeStruct((NC, NPAD, DD), jnp.float32),
    mesh=_mesh,
    compiler_params=_sc_params,
    scratch_types=[
        pltpu.VMEM((32, CHUNK), jnp.int32),
        pltpu.VMEM((CHUNK, DD), jnp.float32),
        pltpu.VMEM((CHUNK, DD), jnp.float32),
        pltpu.VMEM((CHUNK, DD), jnp.float32),
        pltpu.VMEM((CHUNK, DD), jnp.float32),
        pltpu.VMEM_SHARED((NPAD, DD), jnp.float32),
        pltpu.SemaphoreType.DMA,
        pltpu.SemaphoreType.DMA,
        pltpu.SemaphoreType.DMA,
        pltpu.SemaphoreType.DMA,
        pltpu.SemaphoreType.DMA,
    ],
)
def _agg_kernel(hs_hbm, ed_hbm, out_hbm,
                ed_t, gbuf_a, gbuf_b, sbuf_c, sbuf_d,
                acc, sem_ga, sem_gb, sem_sc, sem_sd, sem_ed):
    c = lax.axis_index("c")
    s = lax.axis_index("s")
    wid = c * NS + s

    def zfill(i, _):
        r = i // (DD // LL)
        d = i % (DD // LL)
        sbuf_c[r, pl.ds(d * LL, LL)] = jnp.zeros((LL,), jnp.float32)
        return 0
    lax.fori_loop(0, CHUNK * (DD // LL), zfill, 0)

    rbase = s * RPT

    def zacc(i, _):
        pltpu.sync_copy(sbuf_c, acc.at[pl.ds(rbase + i * CHUNK, CHUNK)])
        return 0
    lax.fori_loop(0, RPT // CHUNK, zacc, 0)
    plsc.subcore_barrier()

    def load_slot(ci, k):
        row = wid * NCHUNKS + ci
        pltpu.async_copy(ed_hbm.at[pl.ds(8 * row, 8)],
                         ed_t.at[pl.ds(8 * k, 8)], sem_ed)

    def wait_slot(k):
        pltpu.make_async_copy(ed_hbm.at[pl.ds(0, 8)],
                              ed_t.at[pl.ds(8 * k, 8)], sem_ed).wait()

    def start_gather(k, gbuf, sem):
        pltpu.async_copy(hs_hbm.at[ed_t.at[8 * k]], gbuf, sem)

    def wait_gather(gbuf, sem):
        pltpu.make_async_copy(hs_hbm.at[pl.ds(0, CHUNK)], gbuf, sem).wait()

    def start_scatter(k, sbuf, sem):
        pltpu.async_copy(sbuf, acc.at[ed_t.at[8 * k + 1]], sem, add=True)

    def wait_scatter(sbuf, sem):
        pltpu.make_async_copy(sbuf, acc.at[pl.ds(0, CHUNK)], sem).wait()

    def scale(k, gbuf, sbuf):
        rvec = jnp.full((LL,), 8 * k + 2, jnp.int32)

        @plsc.parallel_loop(0, CHUNK, unroll=4)
        def _(e):
            b = plsc.bitcast(
                plsc.load_gather(ed_t, [rvec, jnp.full((LL,), e, jnp.int32)]),
                jnp.float32)
            for d in range(DD // LL):
                sbuf[e, pl.ds(d * LL, LL)] = gbuf[e, pl.ds(d * LL, LL)] * b

    def step(ci, k_use, k_load, gbuf, sbuf, gsem, ssem):
        wait_gather(gbuf, gsem)

        @pl.when(ci >= 2)
        def _():
            wait_scatter(sbuf, ssem)

        @pl.when(ci + 2 < NCHUNKS)
        def _():
            load_slot(ci + 2, k_load)

        scale(k_use, gbuf, sbuf)

        @pl.when(ci + 2 < NCHUNKS)
        def _():
            wait_slot(k_load)
            start_gather(k_load, gbuf, gsem)

        start_scatter(k_use, sbuf, ssem)

    load_slot(0, 0)
    load_slot(1, 1)
    wait_slot(0)
    start_gather(0, gbuf_a, sem_ga)
    wait_slot(1)
    start_gather(1, gbuf_b, sem_gb)

    def quad(q, _):
        ci = 4 * q
        step(ci, 0, 2, gbuf_a, sbuf_c, sem_ga, sem_sc)
        step(ci + 1, 1, 3, gbuf_b, sbuf_d, sem_gb, sem_sd)
        step(ci + 2, 2, 0, gbuf_a, sbuf_c, sem_ga, sem_sc)
        step(ci + 3, 3, 1, gbuf_b, sbuf_d, sem_gb, sem_sd)
        return 0
    lax.fori_loop(0, NCHUNKS // 4, quad, 0)

    step(NCHUNKS - 1, 0, 2, gbuf_a, sbuf_c, sem_ga, sem_sc)
    wait_scatter(sbuf_c, sem_sc)
    wait_scatter(sbuf_d, sem_sd)

    plsc.subcore_barrier()
    pltpu.sync_copy(acc.at[pl.ds(rbase, RPT)], out_hbm.at[c, pl.ds(rbase, RPT)])


RB = 400
GRID = NN // RB


def _tc1_body(x_ref, w_ref, d0_ref, d1_ref, o_ref):
    deg = d0_ref[...] + d1_ref[...] + 1.0
    dis = lax.rsqrt(deg)
    h = jnp.dot(x_ref[...], w_ref[...], preferred_element_type=jnp.float32)
    o_ref[...] = h * dis


def _tc2_body(p0_ref, p1_ref, hs_ref, d0_ref, d1_ref, w_ref, b_ref, a_ref,
              o_ref):
    deg = d0_ref[...] + d1_ref[...] + 1.0
    dis = lax.rsqrt(deg)
    t = dis * (p0_ref[...] + p1_ref[...] + hs_ref[...]) + b_ref[...]
    z = jnp.where(t > 0, t, a_ref[...] * t)
    m = jnp.dot(z, w_ref[...], preferred_element_type=jnp.float32)
    o_ref[...] = m * dis


def _tc3_body(p0_ref, p1_ref, hs_ref, d0_ref, d1_ref, b_ref, a_ref, o_ref):
    deg = d0_ref[...] + d1_ref[...] + 1.0
    dis = lax.rsqrt(deg)
    t = dis * (p0_ref[...] + p1_ref[...] + hs_ref[...]) + b_ref[...]
    o_ref[...] = jnp.where(t > 0, t, a_ref[...] * t)


_row_spec = pl.BlockSpec((RB, DD), lambda i: (i, 0))
_col_spec = pl.BlockSpec((RB, 1), lambda i: (i, 0))
_w_spec = pl.BlockSpec((DD, DD), lambda i: (0, 0))
_vec_spec = pl.BlockSpec((1, DD), lambda i: (0, 0))
_out_sds = jax.ShapeDtypeStruct((NN, DD), jnp.float32)

_tc1 = pl.pallas_call(
    _tc1_body, grid=(GRID,),
    in_specs=[_row_spec, _w_spec, _col_spec, _col_spec],
    out_specs=_row_spec, out_shape=_out_sds)

_tc2 = pl.pallas_call(
    _tc2_body, grid=(GRID,),
    in_specs=[_row_spec, _row_spec, _row_spec, _col_spec, _col_spec,
              _w_spec, _vec_spec, _vec_spec],
    out_specs=_row_spec, out_shape=_out_sds)

_tc3 = pl.pallas_call(
    _tc3_body, grid=(GRID,),
    in_specs=[_row_spec, _row_spec, _row_spec, _col_spec, _col_spec,
              _vec_spec, _vec_spec],
    out_specs=_row_spec, out_shape=_out_sds)


def kernel(x, edge_index, edge_weights, W1, b1, W2, b2, alpha):
    src = edge_index[0].astype(jnp.int32)
    dst = edge_index[1].astype(jnp.int32)
    ew = edge_weights.astype(jnp.float32)

    ed = jnp.pad(
        jnp.stack([src.reshape(NW * NCHUNKS, CHUNK),
                   dst.reshape(NW * NCHUNKS, CHUNK),
                   lax.bitcast_convert_type(ew, jnp.int32)
                      .reshape(NW * NCHUNKS, CHUNK)], axis=1),
        ((0, 0), (0, 5), (0, 0))).reshape(NW * NCHUNKS * 8, CHUNK)

    degp = _deg_kernel(dst, ew)
    d0 = degp[0, :NN].reshape(NN, 1)
    d1 = degp[1, :NN].reshape(NN, 1)
    b1r = b1.reshape(1, DD)
    b2r = b2.reshape(1, DD)
    ar = alpha.reshape(1, DD)

    hs1 = _tc1(x, W1, d0, d1)
    p = _agg_kernel(hs1, ed)
    hs2 = _tc2(p[0, :NN], p[1, :NN], hs1, d0, d1, W2, b1r, ar)
    q = _agg_kernel(hs2, ed)
    out = _tc3(q[0, :NN], q[1, :NN], hs2, d0, d1, b2r, ar)
    return out


# speedup vs baseline: 1.8705x; 1.0321x over previous
import functools

import jax
import jax.numpy as jnp
from jax import lax
from jax.experimental import pallas as pl
from jax.experimental.pallas import tpu as pltpu
from jax.experimental.pallas import tpu_sc as plsc

NN = 10000
DD = 128
EE = 320000

NC = 2
NS = 16
LL = 16

NW = NC * NS
EPW = EE // NW
CHUNK = 80
NCHUNKS = EPW // CHUNK
NPAD = 10240
RPT = NPAD // NS

_mesh = plsc.VectorSubcoreMesh(
    core_axis_name="c", subcore_axis_name="s", num_cores=NC, num_subcores=NS)
_sc_params = pltpu.CompilerParams(needs_layout_passes=False)


@functools.partial(
    pl.kernel,
    out_type=jax.ShapeDtypeStruct((NC, NPAD), jnp.float32),
    mesh=_mesh,
    compiler_params=_sc_params,
    scratch_types=[
        pltpu.VMEM((EPW,), jnp.int32),
        pltpu.VMEM((EPW,), jnp.float32),
        pltpu.VMEM((NPAD,), jnp.float32),
        pltpu.VMEM((NS, RPT), jnp.float32),
        pltpu.VMEM((RPT,), jnp.float32),
        pltpu.VMEM_SHARED((NS, NPAD), jnp.float32),
    ],
)
def _deg_kernel(dst_hbm, ew_hbm, out_hbm, dst_v, ew_v, deg_v, red_v, outb_v,
                shared):
    c = lax.axis_index("c")
    s = lax.axis_index("s")
    wid = c * NS + s

    def zero(i, _):
        deg_v[pl.ds(i * LL, LL)] = jnp.zeros((LL,), jnp.float32)
        return 0
    lax.fori_loop(0, NPAD // LL, zero, 0)

    pltpu.sync_copy(dst_hbm.at[pl.ds(wid * EPW, EPW)], dst_v)
    pltpu.sync_copy(ew_hbm.at[pl.ds(wid * EPW, EPW)], ew_v)

    def accum(i, _):
        idx = dst_v[pl.ds(i * LL, LL)]
        w = ew_v[pl.ds(i * LL, LL)]
        plsc.addupdate_scatter(deg_v, [idx], w)
        return 0
    lax.fori_loop(0, EPW // LL, accum, 0)

    pltpu.sync_copy(deg_v, shared.at[s])
    plsc.subcore_barrier()

    base = s * RPT
    pltpu.sync_copy(shared.at[:, pl.ds(base, RPT)], red_v)

    def reduce_vreg(j, _):
        acc = red_v[0, pl.ds(j * LL, LL)]
        for p in range(1, NS):
            acc = acc + red_v[p, pl.ds(j * LL, LL)]
        outb_v[pl.ds(j * LL, LL)] = acc
        return 0
    lax.fori_loop(0, RPT // LL, reduce_vreg, 0)
    pltpu.sync_copy(outb_v, out_hbm.at[c, pl.ds(base, RPT)])


@functools.partial(
    pl.kernel,
    out_type=jax.ShapeDtypeStruct((NC, NPAD, DD), jnp.float32),
    mesh=_mesh,
    compiler_params=_sc_params,
    scratch_types=[
        pltpu.VMEM((32, CHUNK), jnp.int32),
        pltpu.VMEM((CHUNK, DD), jnp.float32),
        pltpu.VMEM((CHUNK, DD), jnp.float32),
        pltpu.VMEM((CHUNK, DD), jnp.float32),
        pltpu.VMEM((CHUNK, DD), jnp.float32),
        pltpu.VMEM_SHARED((NPAD, DD), jnp.float32),
        pltpu.SemaphoreType.DMA,
        pltpu.SemaphoreType.DMA,
        pltpu.SemaphoreType.DMA,
        pltpu.SemaphoreType.DMA,
        pltpu.SemaphoreType.DMA,
    ],
)
def _agg_kernel(hs_hbm, ed_hbm, out_hbm,
                ed_t, gbuf_a, gbuf_b, sbuf_c, sbuf_d,
                acc, sem_ga, sem_gb, sem_sc, sem_sd, sem_ed):
    c = lax.axis_index("c")
    s = lax.axis_index("s")
    wid = c * NS + s

    def zfill(i, _):
        r = i // (DD // LL)
        d = i % (DD // LL)
        sbuf_c[r, pl.ds(d * LL, LL)] = jnp.zeros((LL,), jnp.float32)
        return 0
    lax.fori_loop(0, CHUNK * (DD // LL), zfill, 0)

    rbase = s * RPT

    def zacc(i, _):
        pltpu.sync_copy(sbuf_c, acc.at[pl.ds(rbase + i * CHUNK, CHUNK)])
        return 0
    lax.fori_loop(0, RPT // CHUNK, zacc, 0)
    plsc.subcore_barrier()

    def load_slot(ci, k):
        row = wid * NCHUNKS + ci
        pltpu.async_copy(ed_hbm.at[pl.ds(8 * row, 8)],
                         ed_t.at[pl.ds(8 * k, 8)], sem_ed)

    def wait_slot(k):
        pltpu.make_async_copy(ed_hbm.at[pl.ds(0, 8)],
                              ed_t.at[pl.ds(8 * k, 8)], sem_ed).wait()

    def start_gather(k, gbuf, sem):
        pltpu.async_copy(hs_hbm.at[ed_t.at[8 * k]], gbuf, sem)

    def wait_gather(gbuf, sem):
        pltpu.make_async_copy(hs_hbm.at[pl.ds(0, CHUNK)], gbuf, sem).wait()

    def start_scatter(k, sbuf, sem):
        pltpu.async_copy(sbuf, acc.at[ed_t.at[8 * k + 1]], sem, add=True)

    def wait_scatter(sbuf, sem):
        pltpu.make_async_copy(sbuf, acc.at[pl.ds(0, CHUNK)], sem).wait()

    def scale(k, gbuf, sbuf):
        rvec = jnp.full((LL,), 8 * k + 2, jnp.int32)

        @plsc.parallel_loop(0, CHUNK, unroll=4)
        def _(e):
            b = plsc.bitcast(
                plsc.load_gather(ed_t, [rvec, jnp.full((LL,), e, jnp.int32)]),
                jnp.float32)
            for d in range(DD // LL):
                sbuf[e, pl.ds(d * LL, LL)] = gbuf[e, pl.ds(d * LL, LL)] * b

    def step(ci, k_use, k_load, gbuf, sbuf, gsem, ssem):
        wait_gather(gbuf, gsem)

        @pl.when(ci >= 2)
        def _():
            wait_scatter(sbuf, ssem)

        @pl.when(ci + 2 < NCHUNKS)
        def _():
            load_slot(ci + 2, k_load)

        scale(k_use, gbuf, sbuf)

        @pl.when(ci + 2 < NCHUNKS)
        def _():
            wait_slot(k_load)
            start_gather(k_load, gbuf, gsem)

        start_scatter(k_use, sbuf, ssem)

    load_slot(0, 0)
    load_slot(1, 1)
    wait_slot(0)
    start_gather(0, gbuf_a, sem_ga)
    wait_slot(1)
    start_gather(1, gbuf_b, sem_gb)

    def quad(q, _):
        ci = 4 * q
        step(ci, 0, 2, gbuf_a, sbuf_c, sem_ga, sem_sc)
        step(ci + 1, 1, 3, gbuf_b, sbuf_d, sem_gb, sem_sd)
        step(ci + 2, 2, 0, gbuf_a, sbuf_c, sem_ga, sem_sc)
        step(ci + 3, 3, 1, gbuf_b, sbuf_d, sem_gb, sem_sd)
        return 0
    lax.fori_loop(0, NCHUNKS // 4, quad, 0)

    step(NCHUNKS - 1, 0, 2, gbuf_a, sbuf_c, sem_ga, sem_sc)
    wait_scatter(sbuf_c, sem_sc)
    wait_scatter(sbuf_d, sem_sd)

    plsc.subcore_barrier()
    pltpu.sync_copy(acc.at[pl.ds(rbase, RPT)], out_hbm.at[c, pl.ds(rbase, RPT)])


RB = 400
GRID = NN // RB


def _tc1_body(x_ref, w_ref, d0_ref, d1_ref, o_ref):
    deg = d0_ref[...] + d1_ref[...] + 1.0
    dis = lax.rsqrt(deg)
    h = jnp.dot(x_ref[...], w_ref[...], preferred_element_type=jnp.float32)
    o_ref[...] = h * dis


def _tc2_body(p0_ref, p1_ref, hs_ref, d0_ref, d1_ref, w_ref, b_ref, a_ref,
              o_ref):
    deg = d0_ref[...] + d1_ref[...] + 1.0
    dis = lax.rsqrt(deg)
    t = dis * (p0_ref[0] + p1_ref[0] + hs_ref[...]) + b_ref[...]
    z = jnp.where(t > 0, t, a_ref[...] * t)
    m = jnp.dot(z, w_ref[...], preferred_element_type=jnp.float32)
    o_ref[...] = m * dis


def _tc3_body(p0_ref, p1_ref, hs_ref, d0_ref, d1_ref, b_ref, a_ref, o_ref):
    deg = d0_ref[...] + d1_ref[...] + 1.0
    dis = lax.rsqrt(deg)
    t = dis * (p0_ref[0] + p1_ref[0] + hs_ref[...]) + b_ref[...]
    o_ref[...] = jnp.where(t > 0, t, a_ref[...] * t)


_row_spec = pl.BlockSpec((RB, DD), lambda i: (i, 0))
_col_spec = pl.BlockSpec((RB, 1), lambda i: (i, 0))
_w_spec = pl.BlockSpec((DD, DD), lambda i: (0, 0))
_vec_spec = pl.BlockSpec((1, DD), lambda i: (0, 0))
_p0_spec = pl.BlockSpec((1, RB, DD), lambda i: (0, i, 0))
_p1_spec = pl.BlockSpec((1, RB, DD), lambda i: (1, i, 0))
_out_sds = jax.ShapeDtypeStruct((NN, DD), jnp.float32)

_tc1 = pl.pallas_call(
    _tc1_body, grid=(GRID,),
    in_specs=[_row_spec, _w_spec, _col_spec, _col_spec],
    out_specs=_row_spec, out_shape=_out_sds)

_tc2 = pl.pallas_call(
    _tc2_body, grid=(GRID,),
    in_specs=[_p0_spec, _p1_spec, _row_spec, _col_spec, _col_spec,
              _w_spec, _vec_spec, _vec_spec],
    out_specs=_row_spec, out_shape=_out_sds)

_tc3 = pl.pallas_call(
    _tc3_body, grid=(GRID,),
    in_specs=[_p0_spec, _p1_spec, _row_spec, _col_spec, _col_spec,
              _vec_spec, _vec_spec],
    out_specs=_row_spec, out_shape=_out_sds)


def kernel(x, edge_index, edge_weights, W1, b1, W2, b2, alpha):
    src = edge_index[0].astype(jnp.int32)
    dst = edge_index[1].astype(jnp.int32)
    ew = edge_weights.astype(jnp.float32)

    ed = jnp.pad(
        jnp.stack([src.reshape(NW * NCHUNKS, CHUNK),
                   dst.reshape(NW * NCHUNKS, CHUNK),
                   lax.bitcast_convert_type(ew, jnp.int32)
                      .reshape(NW * NCHUNKS, CHUNK)], axis=1),
        ((0, 0), (0, 5), (0, 0))).reshape(NW * NCHUNKS * 8, CHUNK)

    degp = _deg_kernel(dst, ew)
    d0 = degp[0, :NN].reshape(NN, 1)
    d1 = degp[1, :NN].reshape(NN, 1)
    b1r = b1.reshape(1, DD)
    b2r = b2.reshape(1, DD)
    ar = alpha.reshape(1, DD)

    hs1 = _tc1(x, W1, d0, d1)
    p = _agg_kernel(hs1, ed)
    hs2 = _tc2(p, p, hs1, d0, d1, W2, b1r, ar)
    q = _agg_kernel(hs2, ed)
    out = _tc3(q, q, hs2, d0, d1, b2r, ar)
    return out


# speedup vs baseline: 1.9804x; 1.0587x over previous
import functools

import jax
import jax.numpy as jnp
from jax import lax
from jax.experimental import pallas as pl
from jax.experimental.pallas import tpu as pltpu
from jax.experimental.pallas import tpu_sc as plsc

NN = 10000
DD = 128
EE = 320000

NC = 2
NS = 16
LL = 16

NW = NC * NS
EPW = EE // NW
CHUNK = 80
NCHUNKS = EPW // CHUNK
NPAD = 10240
RPT = NPAD // NS

_mesh = plsc.VectorSubcoreMesh(
    core_axis_name="c", subcore_axis_name="s", num_cores=NC, num_subcores=NS)
_sc_params = pltpu.CompilerParams(needs_layout_passes=False)


@functools.partial(
    pl.kernel,
    out_type=jax.ShapeDtypeStruct((NC, NPAD), jnp.float32),
    mesh=_mesh,
    compiler_params=_sc_params,
    scratch_types=[
        pltpu.VMEM((EPW,), jnp.int32),
        pltpu.VMEM((EPW,), jnp.float32),
        pltpu.VMEM((NPAD,), jnp.float32),
        pltpu.VMEM((NS, RPT), jnp.float32),
        pltpu.VMEM((RPT,), jnp.float32),
        pltpu.VMEM_SHARED((NS, NPAD), jnp.float32),
    ],
)
def _deg_kernel(dst_hbm, ew_hbm, out_hbm, dst_v, ew_v, deg_v, red_v, outb_v,
                shared):
    c = lax.axis_index("c")
    s = lax.axis_index("s")
    wid = c * NS + s

    def zero(i, _):
        deg_v[pl.ds(i * LL, LL)] = jnp.zeros((LL,), jnp.float32)
        return 0
    lax.fori_loop(0, NPAD // LL, zero, 0)

    pltpu.sync_copy(dst_hbm.at[pl.ds(wid * EPW, EPW)], dst_v)
    pltpu.sync_copy(ew_hbm.at[pl.ds(wid * EPW, EPW)], ew_v)

    def accum(i, _):
        idx = dst_v[pl.ds(i * LL, LL)]
        w = ew_v[pl.ds(i * LL, LL)]
        plsc.addupdate_scatter(deg_v, [idx], w)
        return 0
    lax.fori_loop(0, EPW // LL, accum, 0)

    pltpu.sync_copy(deg_v, shared.at[s])
    plsc.subcore_barrier()

    base = s * RPT
    pltpu.sync_copy(shared.at[:, pl.ds(base, RPT)], red_v)

    def reduce_vreg(j, _):
        acc = red_v[0, pl.ds(j * LL, LL)]
        for p in range(1, NS):
            acc = acc + red_v[p, pl.ds(j * LL, LL)]
        outb_v[pl.ds(j * LL, LL)] = acc
        return 0
    lax.fori_loop(0, RPT // LL, reduce_vreg, 0)
    pltpu.sync_copy(outb_v, out_hbm.at[c, pl.ds(base, RPT)])


@functools.partial(
    pl.kernel,
    out_type=jax.ShapeDtypeStruct((NC, NPAD, DD), jnp.float32),
    mesh=_mesh,
    compiler_params=_sc_params,
    scratch_types=[
        pltpu.VMEM((3, CHUNK), jnp.int32),
        pltpu.VMEM((3, CHUNK), jnp.int32),
        pltpu.VMEM((3, CHUNK), jnp.int32),
        pltpu.VMEM((3, CHUNK), jnp.int32),
        pltpu.VMEM((CHUNK, DD), jnp.float32),
        pltpu.VMEM((CHUNK, DD), jnp.float32),
        pltpu.VMEM((CHUNK, DD), jnp.float32),
        pltpu.VMEM((CHUNK, DD), jnp.float32),
        pltpu.VMEM_SHARED((NPAD, DD), jnp.float32),
        pltpu.SemaphoreType.DMA,
        pltpu.SemaphoreType.DMA,
        pltpu.SemaphoreType.DMA,
        pltpu.SemaphoreType.DMA,
        pltpu.SemaphoreType.DMA,
    ],
)
def _agg_kernel(hs_hbm, ed_hbm, out_hbm,
                ed_t0, ed_t1, ed_t2, ed_t3, gbuf_a, gbuf_b, sbuf_c, sbuf_d,
                acc, sem_ga, sem_gb, sem_sc, sem_sd, sem_ed):
    c = lax.axis_index("c")
    s = lax.axis_index("s")
    wid = c * NS + s
    ed_bufs = [ed_t0, ed_t1, ed_t2, ed_t3]

    def zfill(i, _):
        r = i // (DD // LL)
        d = i % (DD // LL)
        sbuf_c[r, pl.ds(d * LL, LL)] = jnp.zeros((LL,), jnp.float32)
        return 0
    lax.fori_loop(0, CHUNK * (DD // LL), zfill, 0)

    rbase = s * RPT

    def zacc(i, _):
        pltpu.sync_copy(sbuf_c, acc.at[pl.ds(rbase + i * CHUNK, CHUNK)])
        return 0
    lax.fori_loop(0, RPT // CHUNK, zacc, 0)
    plsc.subcore_barrier()

    def load_slot(ci, k):
        row = wid * NCHUNKS + ci
        pltpu.async_copy(ed_hbm.at[row], ed_bufs[k], sem_ed)

    def wait_slot(k):
        pltpu.make_async_copy(ed_hbm.at[0], ed_bufs[k], sem_ed).wait()

    def start_gather(k, gbuf, sem):
        pltpu.async_copy(hs_hbm.at[ed_bufs[k].at[0]], gbuf, sem)

    def wait_gather(gbuf, sem):
        pltpu.make_async_copy(hs_hbm.at[pl.ds(0, CHUNK)], gbuf, sem).wait()

    def start_scatter(k, sbuf, sem):
        pltpu.async_copy(sbuf, acc.at[ed_bufs[k].at[1]], sem, add=True)

    def wait_scatter(sbuf, sem):
        pltpu.make_async_copy(sbuf, acc.at[pl.ds(0, CHUNK)], sem).wait()

    def scale(k, gbuf, sbuf):
        rvec = jnp.full((LL,), 2, jnp.int32)
        ed_k = ed_bufs[k]

        @plsc.parallel_loop(0, CHUNK, unroll=8)
        def _(e):
            b = plsc.bitcast(
                plsc.load_gather(ed_k, [rvec, jnp.full((LL,), e, jnp.int32)]),
                jnp.float32)
            for d in range(DD // LL):
                sbuf[e, pl.ds(d * LL, LL)] = gbuf[e, pl.ds(d * LL, LL)] * b

    def step(ci, k_use, k_load, gbuf, sbuf, gsem, ssem):
        wait_gather(gbuf, gsem)

        @pl.when(ci >= 2)
        def _():
            wait_scatter(sbuf, ssem)

        @pl.when(ci + 2 < NCHUNKS)
        def _():
            load_slot(ci + 2, k_load)

        scale(k_use, gbuf, sbuf)

        @pl.when(ci + 2 < NCHUNKS)
        def _():
            wait_slot(k_load)
            start_gather(k_load, gbuf, gsem)

        start_scatter(k_use, sbuf, ssem)

    load_slot(0, 0)
    load_slot(1, 1)
    wait_slot(0)
    start_gather(0, gbuf_a, sem_ga)
    wait_slot(1)
    start_gather(1, gbuf_b, sem_gb)

    def quad(q, _):
        ci = 4 * q
        step(ci, 0, 2, gbuf_a, sbuf_c, sem_ga, sem_sc)
        step(ci + 1, 1, 3, gbuf_b, sbuf_d, sem_gb, sem_sd)
        step(ci + 2, 2, 0, gbuf_a, sbuf_c, sem_ga, sem_sc)
        step(ci + 3, 3, 1, gbuf_b, sbuf_d, sem_gb, sem_sd)
        return 0
    lax.fori_loop(0, NCHUNKS // 4, quad, 0)

    step(NCHUNKS - 1, 0, 2, gbuf_a, sbuf_c, sem_ga, sem_sc)
    wait_scatter(sbuf_c, sem_sc)
    wait_scatter(sbuf_d, sem_sd)

    plsc.subcore_barrier()
    pltpu.sync_copy(acc.at[pl.ds(rbase, RPT)], out_hbm.at[c, pl.ds(rbase, RPT)])


RB = 400
GRID = NN // RB


def _tc1_body(x_ref, w_ref, d0_ref, d1_ref, o_ref):
    deg = d0_ref[...] + d1_ref[...] + 1.0
    dis = lax.rsqrt(deg)
    h = jnp.dot(x_ref[...], w_ref[...], preferred_element_type=jnp.float32)
    o_ref[...] = h * dis


def _tc2_body(p0_ref, p1_ref, hs_ref, d0_ref, d1_ref, w_ref, b_ref, a_ref,
              o_ref):
    deg = d0_ref[...] + d1_ref[...] + 1.0
    dis = lax.rsqrt(deg)
    t = dis * (p0_ref[0] + p1_ref[0] + hs_ref[...]) + b_ref[...]
    z = jnp.where(t > 0, t, a_ref[...] * t)
    m = jnp.dot(z, w_ref[...], preferred_element_type=jnp.float32)
    o_ref[...] = m * dis


def _tc3_body(p0_ref, p1_ref, hs_ref, d0_ref, d1_ref, b_ref, a_ref, o_ref):
    deg = d0_ref[...] + d1_ref[...] + 1.0
    dis = lax.rsqrt(deg)
    t = dis * (p0_ref[0] + p1_ref[0] + hs_ref[...]) + b_ref[...]
    o_ref[...] = jnp.where(t > 0, t, a_ref[...] * t)


_row_spec = pl.BlockSpec((RB, DD), lambda i: (i, 0))
_col_spec = pl.BlockSpec((RB, 1), lambda i: (i, 0))
_w_spec = pl.BlockSpec((DD, DD), lambda i: (0, 0))
_vec_spec = pl.BlockSpec((1, DD), lambda i: (0, 0))
_p0_spec = pl.BlockSpec((1, RB, DD), lambda i: (0, i, 0))
_p1_spec = pl.BlockSpec((1, RB, DD), lambda i: (1, i, 0))
_out_sds = jax.ShapeDtypeStruct((NN, DD), jnp.float32)

_tc1 = pl.pallas_call(
    _tc1_body, grid=(GRID,),
    in_specs=[_row_spec, _w_spec, _col_spec, _col_spec],
    out_specs=_row_spec, out_shape=_out_sds)

_tc2 = pl.pallas_call(
    _tc2_body, grid=(GRID,),
    in_specs=[_p0_spec, _p1_spec, _row_spec, _col_spec, _col_spec,
              _w_spec, _vec_spec, _vec_spec],
    out_specs=_row_spec, out_shape=_out_sds)

_tc3 = pl.pallas_call(
    _tc3_body, grid=(GRID,),
    in_specs=[_p0_spec, _p1_spec, _row_spec, _col_spec, _col_spec,
              _vec_spec, _vec_spec],
    out_specs=_row_spec, out_shape=_out_sds)


def kernel(x, edge_index, edge_weights, W1, b1, W2, b2, alpha):
    src = edge_index[0].astype(jnp.int32)
    dst = edge_index[1].astype(jnp.int32)
    ew = edge_weights.astype(jnp.float32)

    ed = jnp.stack([src.reshape(NW * NCHUNKS, CHUNK),
                    dst.reshape(NW * NCHUNKS, CHUNK),
                    lax.bitcast_convert_type(ew, jnp.int32)
                       .reshape(NW * NCHUNKS, CHUNK)], axis=1)

    degp = _deg_kernel(dst, ew)
    d0 = degp[0, :NN].reshape(NN, 1)
    d1 = degp[1, :NN].reshape(NN, 1)
    b1r = b1.reshape(1, DD)
    b2r = b2.reshape(1, DD)
    ar = alpha.reshape(1, DD)

    hs1 = _tc1(x, W1, d0, d1)
    p = _agg_kernel(hs1, ed)
    hs2 = _tc2(p, p, hs1, d0, d1, W2, b1r, ar)
    q = _agg_kernel(hs2, ed)
    out = _tc3(q, q, hs2, d0, d1, b2r, ar)
    return out


# speedup vs baseline: 2.0653x; 1.0429x over previous
import functools

import jax
import jax.numpy as jnp
from jax import lax
from jax.experimental import pallas as pl
from jax.experimental.pallas import tpu as pltpu
from jax.experimental.pallas import tpu_sc as plsc

NN = 10000
DD = 128
EE = 320000

NC = 2
NS = 16
LL = 16

NW = NC * NS
EPW = EE // NW
CHUNK = 80
NCHUNKS = EPW // CHUNK
NPAD = 10240
RPT = NPAD // NS

_mesh = plsc.VectorSubcoreMesh(
    core_axis_name="c", subcore_axis_name="s", num_cores=NC, num_subcores=NS)
_sc_params = pltpu.CompilerParams(needs_layout_passes=False)


@functools.partial(
    pl.kernel,
    out_type=jax.ShapeDtypeStruct((NC, NPAD), jnp.float32),
    mesh=_mesh,
    compiler_params=_sc_params,
    scratch_types=[
        pltpu.VMEM((EPW,), jnp.int32),
        pltpu.VMEM((EPW,), jnp.float32),
        pltpu.VMEM((NPAD,), jnp.float32),
        pltpu.VMEM((NS, RPT), jnp.float32),
        pltpu.VMEM((RPT,), jnp.float32),
        pltpu.VMEM_SHARED((NS, NPAD), jnp.float32),
    ],
)
def _deg_kernel(dst_hbm, ew_hbm, out_hbm, dst_v, ew_v, deg_v, red_v, outb_v,
                shared):
    c = lax.axis_index("c")
    s = lax.axis_index("s")
    wid = c * NS + s

    def zero(i, _):
        deg_v[pl.ds(i * LL, LL)] = jnp.zeros((LL,), jnp.float32)
        return 0
    lax.fori_loop(0, NPAD // LL, zero, 0)

    pltpu.sync_copy(dst_hbm.at[pl.ds(wid * EPW, EPW)], dst_v)
    pltpu.sync_copy(ew_hbm.at[pl.ds(wid * EPW, EPW)], ew_v)

    def accum(i, _):
        idx = dst_v[pl.ds(i * LL, LL)]
        w = ew_v[pl.ds(i * LL, LL)]
        plsc.addupdate_scatter(deg_v, [idx], w)
        return 0
    lax.fori_loop(0, EPW // LL, accum, 0)

    pltpu.sync_copy(deg_v, shared.at[s])
    plsc.subcore_barrier()

    base = s * RPT
    pltpu.sync_copy(shared.at[:, pl.ds(base, RPT)], red_v)

    def reduce_vreg(j, _):
        acc = red_v[0, pl.ds(j * LL, LL)]
        for p in range(1, NS):
            acc = acc + red_v[p, pl.ds(j * LL, LL)]
        outb_v[pl.ds(j * LL, LL)] = acc
        return 0
    lax.fori_loop(0, RPT // LL, reduce_vreg, 0)
    pltpu.sync_copy(outb_v, out_hbm.at[c, pl.ds(base, RPT)])


@functools.partial(
    pl.kernel,
    out_type=jax.ShapeDtypeStruct((NC, NPAD, DD), jnp.float32),
    mesh=_mesh,
    compiler_params=_sc_params,
    scratch_types=[
        pltpu.VMEM((3, CHUNK), jnp.int32),
        pltpu.VMEM((3, CHUNK), jnp.int32),
        pltpu.VMEM((3, CHUNK), jnp.int32),
        pltpu.VMEM((3, CHUNK), jnp.int32),
        pltpu.VMEM((CHUNK, DD), jnp.float32),
        pltpu.VMEM((CHUNK, DD), jnp.float32),
        pltpu.VMEM((CHUNK, DD), jnp.float32),
        pltpu.VMEM((CHUNK, DD), jnp.float32),
        pltpu.VMEM_SHARED((NPAD, DD), jnp.float32),
        pltpu.SemaphoreType.DMA,
        pltpu.SemaphoreType.DMA,
        pltpu.SemaphoreType.DMA,
        pltpu.SemaphoreType.DMA,
        pltpu.SemaphoreType.DMA,
    ],
)
def _agg_kernel(hs_hbm, ed_hbm, out_hbm,
                ed_t0, ed_t1, ed_t2, ed_t3, gbuf_a, gbuf_b, sbuf_c, sbuf_d,
                acc, sem_ga, sem_gb, sem_sc, sem_sd, sem_ed):
    c = lax.axis_index("c")
    s = lax.axis_index("s")
    wid = c * NS + s
    ed_bufs = [ed_t0, ed_t1, ed_t2, ed_t3]

    def zfill(i, _):
        r = i // (DD // LL)
        d = i % (DD // LL)
        sbuf_c[r, pl.ds(d * LL, LL)] = jnp.zeros((LL,), jnp.float32)
        return 0
    lax.fori_loop(0, CHUNK * (DD // LL), zfill, 0)

    rbase = s * RPT

    def zacc(i, _):
        pltpu.sync_copy(sbuf_c, acc.at[pl.ds(rbase + i * CHUNK, CHUNK)])
        return 0
    lax.fori_loop(0, RPT // CHUNK, zacc, 0)
    plsc.subcore_barrier()

    def load_slot(ci, k):
        row = wid * NCHUNKS + ci
        pltpu.async_copy(ed_hbm.at[row], ed_bufs[k], sem_ed)

    def wait_slot(k):
        pltpu.make_async_copy(ed_hbm.at[0], ed_bufs[k], sem_ed).wait()

    def start_gather(k, gbuf, sem):
        pltpu.async_copy(hs_hbm.at[ed_bufs[k].at[0]], gbuf, sem)

    def wait_gather(gbuf, sem):
        pltpu.make_async_copy(hs_hbm.at[pl.ds(0, CHUNK)], gbuf, sem).wait()

    def start_scatter(k, sbuf, sem):
        pltpu.async_copy(sbuf, acc.at[ed_bufs[k].at[1]], sem, add=True)

    def wait_scatter(sbuf, sem):
        pltpu.make_async_copy(sbuf, acc.at[pl.ds(0, CHUNK)], sem).wait()

    def scale(k, gbuf, sbuf):
        rvec = jnp.full((LL,), 2, jnp.int32)
        ed_k = ed_bufs[k]

        @plsc.parallel_loop(0, CHUNK, unroll=16)
        def _(e):
            b = plsc.bitcast(
                plsc.load_gather(ed_k, [rvec, jnp.full((LL,), e, jnp.int32)]),
                jnp.float32)
            for d in range(DD // LL):
                sbuf[e, pl.ds(d * LL, LL)] = gbuf[e, pl.ds(d * LL, LL)] * b

    def step(ci, k_use, k_load, gbuf, sbuf, gsem, ssem):
        wait_gather(gbuf, gsem)

        @pl.when(ci >= 2)
        def _():
            wait_scatter(sbuf, ssem)

        @pl.when(ci + 2 < NCHUNKS)
        def _():
            load_slot(ci + 2, k_load)

        scale(k_use, gbuf, sbuf)

        @pl.when(ci + 2 < NCHUNKS)
        def _():
            wait_slot(k_load)
            start_gather(k_load, gbuf, gsem)

        start_scatter(k_use, sbuf, ssem)

    load_slot(0, 0)
    load_slot(1, 1)
    wait_slot(0)
    start_gather(0, gbuf_a, sem_ga)
    wait_slot(1)
    start_gather(1, gbuf_b, sem_gb)

    def quad(q, _):
        ci = 4 * q
        step(ci, 0, 2, gbuf_a, sbuf_c, sem_ga, sem_sc)
        step(ci + 1, 1, 3, gbuf_b, sbuf_d, sem_gb, sem_sd)
        step(ci + 2, 2, 0, gbuf_a, sbuf_c, sem_ga, sem_sc)
        step(ci + 3, 3, 1, gbuf_b, sbuf_d, sem_gb, sem_sd)
        return 0
    lax.fori_loop(0, NCHUNKS // 4, quad, 0)

    step(NCHUNKS - 1, 0, 2, gbuf_a, sbuf_c, sem_ga, sem_sc)
    wait_scatter(sbuf_c, sem_sc)
    wait_scatter(sbuf_d, sem_sd)

    plsc.subcore_barrier()
    pltpu.sync_copy(acc.at[pl.ds(rbase, RPT)], out_hbm.at[c, pl.ds(rbase, RPT)])


RB = 400
GRID = NN // RB


def _tc1_body(x_ref, w_ref, d0_ref, d1_ref, o_ref):
    deg = d0_ref[...] + d1_ref[...] + 1.0
    dis = lax.rsqrt(deg)
    h = jnp.dot(x_ref[...], w_ref[...], preferred_element_type=jnp.float32)
    o_ref[...] = h * dis


def _tc2_body(p0_ref, p1_ref, hs_ref, d0_ref, d1_ref, w_ref, b_ref, a_ref,
              o_ref):
    deg = d0_ref[...] + d1_ref[...] + 1.0
    dis = lax.rsqrt(deg)
    t = dis * (p0_ref[0] + p1_ref[0] + hs_ref[...]) + b_ref[...]
    z = jnp.where(t > 0, t, a_ref[...] * t)
    m = jnp.dot(z, w_ref[...], preferred_element_type=jnp.float32)
    o_ref[...] = m * dis


def _tc3_body(p0_ref, p1_ref, hs_ref, d0_ref, d1_ref, b_ref, a_ref, o_ref):
    deg = d0_ref[...] + d1_ref[...] + 1.0
    dis = lax.rsqrt(deg)
    t = dis * (p0_ref[0] + p1_ref[0] + hs_ref[...]) + b_ref[...]
    o_ref[...] = jnp.where(t > 0, t, a_ref[...] * t)


_row_spec = pl.BlockSpec((RB, DD), lambda i: (i, 0))
_col_spec = pl.BlockSpec((RB, 1), lambda i: (i, 0))
_w_spec = pl.BlockSpec((DD, DD), lambda i: (0, 0))
_vec_spec = pl.BlockSpec((1, DD), lambda i: (0, 0))
_p0_spec = pl.BlockSpec((1, RB, DD), lambda i: (0, i, 0))
_p1_spec = pl.BlockSpec((1, RB, DD), lambda i: (1, i, 0))
_out_sds = jax.ShapeDtypeStruct((NN, DD), jnp.float32)

_tc1 = pl.pallas_call(
    _tc1_body, grid=(GRID,),
    in_specs=[_row_spec, _w_spec, _col_spec, _col_spec],
    out_specs=_row_spec, out_shape=_out_sds)

_tc2 = pl.pallas_call(
    _tc2_body, grid=(GRID,),
    in_specs=[_p0_spec, _p1_spec, _row_spec, _col_spec, _col_spec,
              _w_spec, _vec_spec, _vec_spec],
    out_specs=_row_spec, out_shape=_out_sds)

_tc3 = pl.pallas_call(
    _tc3_body, grid=(GRID,),
    in_specs=[_p0_spec, _p1_spec, _row_spec, _col_spec, _col_spec,
              _vec_spec, _vec_spec],
    out_specs=_row_spec, out_shape=_out_sds)


def kernel(x, edge_index, edge_weights, W1, b1, W2, b2, alpha):
    src = edge_index[0].astype(jnp.int32)
    dst = edge_index[1].astype(jnp.int32)
    ew = edge_weights.astype(jnp.float32)

    ed = jnp.stack([src.reshape(NW * NCHUNKS, CHUNK),
                    dst.reshape(NW * NCHUNKS, CHUNK),
                    lax.bitcast_convert_type(ew, jnp.int32)
                       .reshape(NW * NCHUNKS, CHUNK)], axis=1)

    degp = _deg_kernel(dst, ew)
    d0 = degp[0, :NN].reshape(NN, 1)
    d1 = degp[1, :NN].reshape(NN, 1)
    b1r = b1.reshape(1, DD)
    b2r = b2.reshape(1, DD)
    ar = alpha.reshape(1, DD)

    hs1 = _tc1(x, W1, d0, d1)
    p = _agg_kernel(hs1, ed)
    hs2 = _tc2(p, p, hs1, d0, d1, W2, b1r, ar)
    q = _agg_kernel(hs2, ed)
    out = _tc3(q, q, hs2, d0, d1, b2r, ar)
    return out
